# baseline (device time: 22520 ns/iter reference)
import functools

import jax
import jax.numpy as jnp
from jax import lax
from jax.experimental import pallas as pl
from jax.experimental.pallas import tpu as pltpu

N_DEV = 8
B = 2
SQ = 256
SKV_LOC = 256
HQ = 4
DH = 64
D_MODEL = 512
BLK = 64
SCALE = 0.125
NEG = -1e9
LOG2E = 1.4426950408889634

XOR_MASKS = (1, 3, 4)
N_STEPS = len(XOR_MASKS)

N_CHUNKS = 4
CW = 2 * DH


def kernel(x, Wq, K_ext, V_ext, Wo):
    def body(
        x_hbm,
        wq_hbm,
        k_hbm,
        v_hbm,
        wo_hbm,
        out_hbm,
        x_v,
        wq_v,
        k_v,
        v_v,
        wo_v,
        out_v,
        ctx_send,
        ctx_recv,
        l_send,
        l_recv,
        in_sems,
        out_sem,
        ctx_send_sems,
        ctx_recv_sems,
        l_send_sems,
        l_recv_sems,
    ):
        my = lax.axis_index("i")

        in_copies = []
        for i, (src, dst) in enumerate(
            [(x_hbm, x_v), (wq_hbm, wq_v), (k_hbm, k_v), (v_hbm, v_v), (wo_hbm, wo_v)]
        ):
            dma = pltpu.make_async_copy(src, dst, in_sems.at[i])
            dma.start()
            in_copies.append(dma)

        barrier = pltpu.get_barrier_semaphore()
        for m in XOR_MASKS:
            partner = jnp.bitwise_xor(my, m)
            pl.semaphore_signal(
                barrier,
                inc=1,
                device_id=(partner,),
                device_id_type=pl.DeviceIdType.MESH,
            )
        pl.semaphore_wait(barrier, N_STEPS)

        def ctx_rdma(c, s):
            partner = jnp.bitwise_xor(my, XOR_MASKS[s])
            return pltpu.make_async_remote_copy(
                src_ref=ctx_send.at[c, s],
                dst_ref=ctx_recv.at[c, s],
                send_sem=ctx_send_sems.at[c, s],
                recv_sem=ctx_recv_sems.at[c, s],
                device_id=(partner,),
                device_id_type=pl.DeviceIdType.MESH,
            )

        def l_rdma(s):
            partner = jnp.bitwise_xor(my, XOR_MASKS[s])
            return pltpu.make_async_remote_copy(
                src_ref=l_send.at[s],
                dst_ref=l_recv.at[s],
                send_sem=l_send_sems.at[s],
                recv_sem=l_recv_sems.at[s],
                device_id=(partner,),
                device_id_type=pl.DeviceIdType.MESH,
            )

        qb = lax.broadcasted_iota(jnp.int32, (SQ, SKV_LOC), 0) // BLK
        kb = my * (SKV_LOC // BLK) + lax.broadcasted_iota(
            jnp.int32, (SQ, SKV_LOC), 1
        ) // BLK
        mask = (qb == kb) | (kb == 0) | (((qb + kb) % 3) == 0)

        in_copies[0].wait()
        in_copies[1].wait()
        q_all = jnp.dot(
            x_v[:].reshape(B * SQ, D_MODEL),
            wq_v[:],
            preferred_element_type=jnp.float32,
        )

        in_copies[2].wait()
        in_copies[3].wait()
        acc = [None] * N_CHUNKS
        l_rows = [None] * (B * HQ)
        ctx_r = [[None] * N_STEPS for _ in range(N_CHUNKS)]
        for c in range(N_CHUNKS):
            b, hg = divmod(c, 2)
            cols = []
            for h in (2 * hg, 2 * hg + 1):
                q_bh = q_all[b * SQ : (b + 1) * SQ, h * DH : (h + 1) * DH]
                s = jnp.dot(
                    q_bh, k_v[b, h], preferred_element_type=jnp.float32
                )
                w = jnp.exp2(jnp.where(mask, s * (SCALE * LOG2E), NEG))
                l_rows[b * HQ + h] = jnp.sum(w, axis=1)
                cols.append(
                    lax.dot_general(
                        w,
                        v_v[b, h],
                        (((1,), (1,)), ((), ())),
                        preferred_element_type=jnp.float32,
                    )
                )
            acc[c] = jnp.concatenate(cols, axis=1)
            ctx_send[c, 0] = acc[c].astype(jnp.bfloat16)
            ctx_r[c][0] = ctx_rdma(c, 0)
            ctx_r[c][0].start()

        acc_l = jnp.stack(l_rows)
        l_send[0] = acc_l
        rl = l_rdma(0)
        rl.start()

        for s in range(N_STEPS):
            for c in range(N_CHUNKS):
                ctx_r[c][s].wait()
                acc[c] = acc[c] + ctx_recv[c, s].astype(jnp.float32)
                if s + 1 < N_STEPS:
                    ctx_send[c, s + 1] = acc[c].astype(jnp.bfloat16)
                    ctx_r[c][s + 1] = ctx_rdma(c, s + 1)
                    ctx_r[c][s + 1].start()
            rl.wait()
            acc_l = acc_l + l_recv[s]
            if s + 1 < N_STEPS:
                l_send[s + 1] = acc_l
                rl = l_rdma(s + 1)
                rl.start()

        recip = 1.0 / acc_l
        flat_rows = []
        for b in range(B):
            cols = []
            for h in range(HQ):
                c, sub = divmod(h, 2)
                blk = acc[b * 2 + c][:, sub * DH : (sub + 1) * DH]
                r = recip[b * HQ + h, :][:, None]
                cols.append(blk * r)
            flat_rows.append(jnp.concatenate(cols, axis=1))
        flat = jnp.concatenate(flat_rows, axis=0)
        in_copies[4].wait()
        out = jnp.dot(flat, wo_v[:], preferred_element_type=jnp.float32)
        out_v[:] = out.reshape(B, SQ, D_MODEL)

        out_dma = pltpu.make_async_copy(out_v, out_hbm, out_sem)
        out_dma.start()

        @functools.partial(
            pl.run_scoped, second_barrier=pltpu.SemaphoreType.REGULAR
        )
        def _(second_barrier):
            for m in XOR_MASKS:
                partner = jnp.bitwise_xor(my, m)
                pl.semaphore_signal(
                    second_barrier,
                    inc=1,
                    device_id=(partner,),
                    device_id_type=pl.DeviceIdType.MESH,
                )
            pl.semaphore_wait(second_barrier, N_STEPS)

        out_dma.wait()

    k_t = lax.transpose(K_ext, (0, 2, 3, 1))
    v_t = lax.transpose(V_ext, (0, 2, 3, 1))

    return pl.pallas_call(
        body,
        out_shape=jax.ShapeDtypeStruct((B, SQ, D_MODEL), jnp.float32),
        in_specs=[pl.BlockSpec(memory_space=pl.ANY)] * 5,
        out_specs=pl.BlockSpec(memory_space=pl.ANY),
        scratch_shapes=[
            pltpu.VMEM((B, SQ, D_MODEL), jnp.float32),
            pltpu.VMEM((D_MODEL, HQ * DH), jnp.float32),
            pltpu.VMEM((B, HQ, DH, SKV_LOC), jnp.float32),
            pltpu.VMEM((B, HQ, DH, SKV_LOC), jnp.float32),
            pltpu.VMEM((HQ * DH, D_MODEL), jnp.float32),
            pltpu.VMEM((B, SQ, D_MODEL), jnp.float32),
            pltpu.VMEM((N_CHUNKS, N_STEPS, SQ, CW), jnp.bfloat16),
            pltpu.VMEM((N_CHUNKS, N_STEPS, SQ, CW), jnp.bfloat16),
            pltpu.VMEM((N_STEPS, B * HQ, SQ), jnp.float32),
            pltpu.VMEM((N_STEPS, B * HQ, SQ), jnp.float32),
            pltpu.SemaphoreType.DMA((5,)),
            pltpu.SemaphoreType.DMA,
            pltpu.SemaphoreType.DMA((N_CHUNKS, N_STEPS)),
            pltpu.SemaphoreType.DMA((N_CHUNKS, N_STEPS)),
            pltpu.SemaphoreType.DMA((N_STEPS,)),
            pltpu.SemaphoreType.DMA((N_STEPS,)),
        ],
        compiler_params=pltpu.CompilerParams(collective_id=0),
    )(x, Wq, k_t, v_t, Wo)


# device time: 20588 ns/iter; 1.0938x vs baseline; 1.0938x over previous
import functools

import jax
import jax.numpy as jnp
from jax import lax
from jax.experimental import pallas as pl
from jax.experimental.pallas import tpu as pltpu

N_DEV = 8
B = 2
SQ = 256
SKV_LOC = 256
HQ = 4
DH = 64
D_MODEL = 512
BLK = 64
SCALE = 0.125
NEG = -1e9
LOG2E = 1.4426950408889634

XOR_MASKS = (1, 3, 4)
N_STEPS = len(XOR_MASKS)

N_CHUNKS = 4
CW = 2 * DH


def kernel(x, Wq, K_ext, V_ext, Wo):
    def body(
        x_ref,
        wq_ref,
        k_ref,
        v_ref,
        wo_ref,
        out_ref,
        ctx_send,
        ctx_recv,
        l_send,
        l_recv,
        ctx_send_sems,
        ctx_recv_sems,
        l_send_sems,
        l_recv_sems,
    ):
        my = lax.axis_index("i")

        barrier = pltpu.get_barrier_semaphore()
        for m in XOR_MASKS:
            partner = jnp.bitwise_xor(my, m)
            pl.semaphore_signal(
                barrier,
                inc=1,
                device_id=(partner,),
                device_id_type=pl.DeviceIdType.MESH,
            )
        pl.semaphore_wait(barrier, N_STEPS)

        def ctx_rdma(c, s):
            partner = jnp.bitwise_xor(my, XOR_MASKS[s])
            return pltpu.make_async_remote_copy(
                src_ref=ctx_send.at[c, s],
                dst_ref=ctx_recv.at[c, s],
                send_sem=ctx_send_sems.at[c, s],
                recv_sem=ctx_recv_sems.at[c, s],
                device_id=(partner,),
                device_id_type=pl.DeviceIdType.MESH,
            )

        def l_rdma(s):
            partner = jnp.bitwise_xor(my, XOR_MASKS[s])
            return pltpu.make_async_remote_copy(
                src_ref=l_send.at[s],
                dst_ref=l_recv.at[s],
                send_sem=l_send_sems.at[s],
                recv_sem=l_recv_sems.at[s],
                device_id=(partner,),
                device_id_type=pl.DeviceIdType.MESH,
            )

        q_all = jnp.dot(
            x_ref[:].reshape(B * SQ, D_MODEL),
            wq_ref[:],
            preferred_element_type=jnp.float32,
        )

        qb = lax.broadcasted_iota(jnp.int32, (SQ, SKV_LOC), 0) // BLK
        kb = my * (SKV_LOC // BLK) + lax.broadcasted_iota(
            jnp.int32, (SQ, SKV_LOC), 1
        ) // BLK
        mask = (qb == kb) | (kb == 0) | (((qb + kb) % 3) == 0)

        acc = [None] * N_CHUNKS
        l_rows = [None] * (B * HQ)
        ctx_r = [[None] * N_STEPS for _ in range(N_CHUNKS)]
        for c in range(N_CHUNKS):
            b, hg = divmod(c, 2)
            cols = []
            for h in (2 * hg, 2 * hg + 1):
                q_bh = q_all[b * SQ : (b + 1) * SQ, h * DH : (h + 1) * DH]
                s = jnp.dot(
                    q_bh, k_ref[b, h], preferred_element_type=jnp.float32
                )
                w = jnp.exp2(jnp.where(mask, s * (SCALE * LOG2E), NEG))
                l_rows[b * HQ + h] = jnp.sum(w, axis=1)
                cols.append(
                    jnp.dot(
                        w,
                        v_ref[b, :, h, :],
                        preferred_element_type=jnp.float32,
                    )
                )
            acc[c] = jnp.concatenate(cols, axis=1)
            ctx_send[c, 0] = acc[c].astype(jnp.bfloat16)
            ctx_r[c][0] = ctx_rdma(c, 0)
            ctx_r[c][0].start()

        acc_l = jnp.stack(l_rows)
        l_send[0] = acc_l
        rl = l_rdma(0)
        rl.start()

        for s in range(N_STEPS):
            for c in range(N_CHUNKS):
                ctx_r[c][s].wait()
                acc[c] = acc[c] + ctx_recv[c, s].astype(jnp.float32)
                if s + 1 < N_STEPS:
                    ctx_send[c, s + 1] = acc[c].astype(jnp.bfloat16)
                    ctx_r[c][s + 1] = ctx_rdma(c, s + 1)
                    ctx_r[c][s + 1].start()
            rl.wait()
            acc_l = acc_l + l_recv[s]
            if s + 1 < N_STEPS:
                l_send[s + 1] = acc_l
                rl = l_rdma(s + 1)
                rl.start()

        recip = 1.0 / acc_l
        flat_rows = []
        for b in range(B):
            cols = []
            for h in range(HQ):
                c, sub = divmod(h, 2)
                blk = acc[b * 2 + c][:, sub * DH : (sub + 1) * DH]
                r = recip[b * HQ + h, :][:, None]
                cols.append(blk * r)
            flat_rows.append(jnp.concatenate(cols, axis=1))
        flat = jnp.concatenate(flat_rows, axis=0)
        out = jnp.dot(flat, wo_ref[:], preferred_element_type=jnp.float32)
        out_ref[:] = out.reshape(B, SQ, D_MODEL)

        @functools.partial(
            pl.run_scoped, second_barrier=pltpu.SemaphoreType.REGULAR
        )
        def _(second_barrier):
            for m in XOR_MASKS:
                partner = jnp.bitwise_xor(my, m)
                pl.semaphore_signal(
                    second_barrier,
                    inc=1,
                    device_id=(partner,),
                    device_id_type=pl.DeviceIdType.MESH,
                )
            pl.semaphore_wait(second_barrier, N_STEPS)

    k_t = lax.transpose(K_ext, (0, 2, 3, 1))

    return pl.pallas_call(
        body,
        out_shape=jax.ShapeDtypeStruct((B, SQ, D_MODEL), jnp.float32),
        in_specs=[pl.BlockSpec(memory_space=pltpu.VMEM)] * 5,
        out_specs=pl.BlockSpec(memory_space=pltpu.VMEM),
        scratch_shapes=[
            pltpu.VMEM((N_CHUNKS, N_STEPS, SQ, CW), jnp.bfloat16),
            pltpu.VMEM((N_CHUNKS, N_STEPS, SQ, CW), jnp.bfloat16),
            pltpu.VMEM((N_STEPS, B * HQ, SQ), jnp.float32),
            pltpu.VMEM((N_STEPS, B * HQ, SQ), jnp.float32),
            pltpu.SemaphoreType.DMA((N_CHUNKS, N_STEPS)),
            pltpu.SemaphoreType.DMA((N_CHUNKS, N_STEPS)),
            pltpu.SemaphoreType.DMA((N_STEPS,)),
            pltpu.SemaphoreType.DMA((N_STEPS,)),
        ],
        compiler_params=pltpu.CompilerParams(collective_id=0),
    )(x, Wq, k_t, V_ext, Wo)


# device time: 19780 ns/iter; 1.1385x vs baseline; 1.0408x over previous
import functools

import jax
import jax.numpy as jnp
from jax import lax
from jax.experimental import pallas as pl
from jax.experimental.pallas import tpu as pltpu

N_DEV = 8
B = 2
SQ = 256
SKV_LOC = 256
HQ = 4
DH = 64
D_MODEL = 512
BLK = 64
SCALE = 0.125
NEG = -1e9
LOG2E = 1.4426950408889634

XOR_MASKS = (1, 3, 4)
N_STEPS = len(XOR_MASKS)

N_CHUNKS = 4
CW = 2 * DH


def kernel(x, Wq, K_ext, V_ext, Wo):
    def body(
        x_ref,
        wq_ref,
        k_ref,
        v_ref,
        wo_ref,
        out_ref,
        out_v,
        ctx_send,
        ctx_recv,
        l_send,
        l_recv,
        out_sem,
        ctx_send_sems,
        ctx_recv_sems,
        l_send_sems,
        l_recv_sems,
    ):
        my = lax.axis_index("i")

        barrier = pltpu.get_barrier_semaphore()
        for m in XOR_MASKS:
            partner = jnp.bitwise_xor(my, m)
            pl.semaphore_signal(
                barrier,
                inc=1,
                device_id=(partner,),
                device_id_type=pl.DeviceIdType.MESH,
            )
        pl.semaphore_wait(barrier, N_STEPS)

        def ctx_rdma(c, s):
            partner = jnp.bitwise_xor(my, XOR_MASKS[s])
            return pltpu.make_async_remote_copy(
                src_ref=ctx_send.at[c, s],
                dst_ref=ctx_recv.at[c, s],
                send_sem=ctx_send_sems.at[c, s],
                recv_sem=ctx_recv_sems.at[c, s],
                device_id=(partner,),
                device_id_type=pl.DeviceIdType.MESH,
            )

        def l_rdma(s):
            partner = jnp.bitwise_xor(my, XOR_MASKS[s])
            return pltpu.make_async_remote_copy(
                src_ref=l_send.at[s],
                dst_ref=l_recv.at[s],
                send_sem=l_send_sems.at[s],
                recv_sem=l_recv_sems.at[s],
                device_id=(partner,),
                device_id_type=pl.DeviceIdType.MESH,
            )

        q_all = jnp.dot(
            x_ref[:].reshape(B * SQ, D_MODEL),
            wq_ref[:],
            preferred_element_type=jnp.float32,
        )

        qb = lax.broadcasted_iota(jnp.int32, (SQ, SKV_LOC), 0) // BLK
        kb = my * (SKV_LOC // BLK) + lax.broadcasted_iota(
            jnp.int32, (SQ, SKV_LOC), 1
        ) // BLK
        mask = (qb == kb) | (kb == 0) | (((qb + kb) % 3) == 0)

        acc = [None] * N_CHUNKS
        l_rows = [None] * (B * HQ)
        ctx_r = [[None] * N_STEPS for _ in range(N_CHUNKS)]
        for c in range(N_CHUNKS):
            b, hg = divmod(c, 2)
            cols = []
            for h in (2 * hg, 2 * hg + 1):
                q_bh = q_all[b * SQ : (b + 1) * SQ, h * DH : (h + 1) * DH]
                s = jnp.dot(
                    q_bh, k_ref[b, h], preferred_element_type=jnp.float32
                )
                w = jnp.exp2(jnp.where(mask, s * (SCALE * LOG2E), NEG))
                l_rows[b * HQ + h] = jnp.sum(w, axis=1)
                cols.append(
                    jnp.dot(
                        w,
                        v_ref[b, :, h, :],
                        preferred_element_type=jnp.float32,
                    )
                )
            acc[c] = jnp.concatenate(cols, axis=1)
            ctx_send[c, 0] = acc[c].astype(jnp.bfloat16)
            ctx_r[c][0] = ctx_rdma(c, 0)
            ctx_r[c][0].start()

        acc_l = jnp.stack(l_rows)
        l_send[0] = acc_l
        rl = l_rdma(0)
        rl.start()

        for s in range(N_STEPS):
            for c in range(N_CHUNKS):
                ctx_r[c][s].wait()
                acc[c] = acc[c] + ctx_recv[c, s].astype(jnp.float32)
                if s + 1 < N_STEPS:
                    ctx_send[c, s + 1] = acc[c].astype(jnp.bfloat16)
                    ctx_r[c][s + 1] = ctx_rdma(c, s + 1)
                    ctx_r[c][s + 1].start()
            rl.wait()
            acc_l = acc_l + l_recv[s]
            if s + 1 < N_STEPS:
                l_send[s + 1] = acc_l
                rl = l_rdma(s + 1)
                rl.start()

        recip = 1.0 / acc_l
        flat_rows = []
        for b in range(B):
            cols = []
            for h in range(HQ):
                c, sub = divmod(h, 2)
                blk = acc[b * 2 + c][:, sub * DH : (sub + 1) * DH]
                r = recip[b * HQ + h, :][:, None]
                cols.append(blk * r)
            flat_rows.append(jnp.concatenate(cols, axis=1))
        flat = jnp.concatenate(flat_rows, axis=0)
        out = jnp.dot(flat, wo_ref[:], preferred_element_type=jnp.float32)
        out_v[:] = out.reshape(B, SQ, D_MODEL)

        out_dma = pltpu.make_async_copy(out_v, out_ref, out_sem)
        out_dma.start()

        @functools.partial(
            pl.run_scoped, second_barrier=pltpu.SemaphoreType.REGULAR
        )
        def _(second_barrier):
            for m in XOR_MASKS:
                partner = jnp.bitwise_xor(my, m)
                pl.semaphore_signal(
                    second_barrier,
                    inc=1,
                    device_id=(partner,),
                    device_id_type=pl.DeviceIdType.MESH,
                )
            pl.semaphore_wait(second_barrier, N_STEPS)

        out_dma.wait()

    k_t = lax.transpose(K_ext, (0, 2, 3, 1))

    return pl.pallas_call(
        body,
        out_shape=jax.ShapeDtypeStruct((B, SQ, D_MODEL), jnp.float32),
        in_specs=[pl.BlockSpec(memory_space=pltpu.VMEM)] * 5,
        out_specs=pl.BlockSpec(memory_space=pl.ANY),
        scratch_shapes=[
            pltpu.VMEM((B, SQ, D_MODEL), jnp.float32),
            pltpu.VMEM((N_CHUNKS, N_STEPS, SQ, CW), jnp.bfloat16),
            pltpu.VMEM((N_CHUNKS, N_STEPS, SQ, CW), jnp.bfloat16),
            pltpu.VMEM((N_STEPS, B * HQ, SQ), jnp.float32),
            pltpu.VMEM((N_STEPS, B * HQ, SQ), jnp.float32),
            pltpu.SemaphoreType.DMA,
            pltpu.SemaphoreType.DMA((N_CHUNKS, N_STEPS)),
            pltpu.SemaphoreType.DMA((N_CHUNKS, N_STEPS)),
            pltpu.SemaphoreType.DMA((N_STEPS,)),
            pltpu.SemaphoreType.DMA((N_STEPS,)),
        ],
        compiler_params=pltpu.CompilerParams(collective_id=0),
    )(x, Wq, k_t, V_ext, Wo)
